# baseline (speedup 1.0000x reference)
import functools

import jax
import jax.numpy as jnp
from jax import lax
from jax.experimental import pallas as pl
from jax.experimental.pallas import tpu as pltpu
from jax.experimental.pallas import tpu_sc as plsc

VOCAB = 100000
DIM = 50
CTX = 200

ROWS_PER_TILE = 8
N_ACTIVE = CTX // ROWS_PER_TILE

NSTREAM = 5
SUB = 2000
NROWS = VOCAB // SUB
NSTEP = NROWS // NSTREAM


def _sc_gather(idx, table):
    info = plsc.get_sparse_core_info()
    nc = info.num_cores

    mesh = plsc.VectorSubcoreMesh(core_axis_name="c", subcore_axis_name="s")

    @functools.partial(
        pl.kernel,
        mesh=mesh,
        compiler_params=pltpu.CompilerParams(needs_layout_passes=False),
        out_type=jax.ShapeDtypeStruct((CTX, DIM), jnp.float32),
        scratch_types=[
            pltpu.VMEM((16,), jnp.int32),
            pltpu.VMEM((ROWS_PER_TILE, DIM), jnp.float32),
            pltpu.SemaphoreType.DMA,
        ],
    )
    def gather_kernel(idx_hbm, table_hbm, out_hbm, idx_v, rows_v, sem):
        wid = lax.axis_index("s") * nc + lax.axis_index("c")

        @pl.when(wid < N_ACTIVE)
        def _():
            base = wid * ROWS_PER_TILE
            pltpu.sync_copy(idx_hbm.at[pl.ds(base, ROWS_PER_TILE)],
                            idx_v.at[pl.ds(0, ROWS_PER_TILE)])
            lane = lax.iota(jnp.int32, 16)
            idxs = idx_v[...]
            copies = []
            for k in range(ROWS_PER_TILE):
                r = jnp.max(jnp.where(lane == k, idxs, 0))
                copies.append(pltpu.async_copy(
                    table_hbm.at[pl.ds(r, 1)], rows_v.at[pl.ds(k, 1)], sem))
            for c in copies:
                c.wait()
            pltpu.sync_copy(rows_v, out_hbm.at[pl.ds(base, ROWS_PER_TILE)])

    return gather_kernel(idx, table)


def _tc_head_body(g_ref, b_ref, *rest):
    w_refs = rest[:NSTREAM]
    out_ref, pooled, m_ref = rest[NSTREAM:]
    i = pl.program_id(0)

    @pl.when(i == 0)
    def _():
        pooled[...] = jnp.sum(g_ref[...], axis=0, keepdims=True) * (1.0 / CTX)
        m_ref[0] = -jnp.inf

    m = m_ref[0]
    for k in range(NSTREAM):
        row = i * NSTREAM + k
        logits = lax.dot_general(
            pooled[...], w_refs[k][...],
            (((1,), (1,)), ((), ())),
            preferred_element_type=jnp.float32,
        ) + b_ref[pl.ds(row, 1), :]
        out_ref[pl.ds(row, 1), :] = logits
        m = jnp.maximum(m, jnp.max(logits))
    m_ref[0] = m

    @pl.when(i == NSTEP - 1)
    def _():
        mx = m_ref[0]
        lse = mx + jnp.log(jnp.sum(jnp.exp(out_ref[...] - mx)))
        out_ref[...] = out_ref[...] - lse


def _tc_head(gathered, W, b2d, interpret=False):
    w_specs = [
        pl.BlockSpec((SUB, DIM), lambda i, k=k: (NSTREAM * i + k, 0))
        for k in range(NSTREAM)
    ]
    return pl.pallas_call(
        _tc_head_body,
        grid=(NSTEP,),
        in_specs=[
            pl.BlockSpec((CTX, DIM), lambda i: (0, 0)),
            pl.BlockSpec((NROWS, SUB), lambda i: (0, 0)),
        ] + w_specs,
        out_specs=pl.BlockSpec((NROWS, SUB), lambda i: (0, 0)),
        out_shape=jax.ShapeDtypeStruct((NROWS, SUB), jnp.float32),
        scratch_shapes=[
            pltpu.VMEM((1, DIM), jnp.float32),
            pltpu.SMEM((1,), jnp.float32),
        ],
        compiler_params=pltpu.CompilerParams(
            dimension_semantics=("arbitrary",)),
        interpret=interpret,
    )(gathered, b2d, *([W] * NSTREAM))


def kernel(inputs, table, W, b):
    idx = inputs.astype(jnp.int32)
    gathered = _sc_gather(idx, table)
    out = _tc_head(gathered, W, b.reshape(NROWS, SUB))
    return out.reshape(1, VOCAB)


# speedup vs baseline: 3.1750x; 3.1750x over previous
import functools

import jax
import jax.numpy as jnp
from jax import lax
from jax.experimental import pallas as pl
from jax.experimental.pallas import tpu as pltpu
from jax.experimental.pallas import tpu_sc as plsc

VOCAB = 100000
DIM = 50
CTX = 200
DPAD = 64

ROWS_PER_TILE = 8
N_ACTIVE = CTX // ROWS_PER_TILE


def _sc_gather_pool(idx, tableT):
    info = plsc.get_sparse_core_info()
    nc = info.num_cores

    mesh = plsc.VectorSubcoreMesh(core_axis_name="c", subcore_axis_name="s")

    @functools.partial(
        pl.kernel,
        mesh=mesh,
        compiler_params=pltpu.CompilerParams(needs_layout_passes=False),
        out_type=jax.ShapeDtypeStruct((N_ACTIVE, DPAD), jnp.float32),
        scratch_types=[pltpu.VMEM((16,), jnp.int32)]
        + [pltpu.VMEM((DIM, 128), jnp.float32) for _ in range(ROWS_PER_TILE)]
        + [pltpu.VMEM((DPAD,), jnp.float32), pltpu.SemaphoreType.DMA],
    )
    def gather_kernel(idx_hbm, table_hbm, out_hbm, idx_v, *rest):
        slabs = rest[:ROWS_PER_TILE]
        acc_v, sem = rest[ROWS_PER_TILE:]
        wid = lax.axis_index("s") * nc + lax.axis_index("c")

        @pl.when(wid < N_ACTIVE)
        def _():
            base = wid * ROWS_PER_TILE
            pltpu.sync_copy(idx_hbm.at[pl.ds(base, ROWS_PER_TILE)],
                            idx_v.at[pl.ds(0, ROWS_PER_TILE)])
            lane = lax.iota(jnp.int32, 16)
            idxs = idx_v[...]
            rs = []
            copies = []
            for k in range(ROWS_PER_TILE):
                r = jnp.max(jnp.where(lane == k, idxs, 0))
                rs.append(r)
                t = lax.shift_right_logical(r, 7)
                copies.append(pltpu.async_copy(
                    table_hbm.at[:, pl.ds(t * 128, 128)], slabs[k], sem))
            for c in copies:
                c.wait()
            accs = [jnp.zeros((16,), jnp.float32) for _ in range(4)]
            for k in range(ROWS_PER_TILE):
                col = jnp.full((16,), rs[k] & 127, jnp.int32)
                for q in range(4):
                    rows = lane + (16 * q)
                    if 16 * (q + 1) > DIM:
                        valid = rows < DIM
                        rows = jnp.minimum(rows, DIM - 1)
                        g = plsc.load_gather(slabs[k], [rows, col])
                        g = jnp.where(valid, g, 0.0)
                    else:
                        g = plsc.load_gather(slabs[k], [rows, col])
                    accs[q] = accs[q] + g
            for q in range(4):
                acc_v[pl.ds(16 * q, 16)] = accs[q]
            pltpu.sync_copy(acc_v, out_hbm.at[wid])

    return gather_kernel(idx, tableT)


def _tc_head_body(g_ref, wt_ref, b_ref, out_ref):
    pooled = jnp.sum(g_ref[...], axis=0, keepdims=True) * (1.0 / CTX)
    logits = lax.dot_general(
        pooled[:, :DIM], wt_ref[...],
        (((1,), (0,)), ((), ())),
        preferred_element_type=jnp.float32,
    ) + b_ref[...]
    m = jnp.max(logits)
    lse = m + jnp.log(jnp.sum(jnp.exp(logits - m)))
    out_ref[...] = logits - lse


def _tc_head(partials, Wt, b2d, interpret=False):
    return pl.pallas_call(
        _tc_head_body,
        in_specs=[
            pl.BlockSpec((N_ACTIVE, DPAD), lambda: (0, 0)),
            pl.BlockSpec((DIM, VOCAB), lambda: (0, 0)),
            pl.BlockSpec((1, VOCAB), lambda: (0, 0)),
        ],
        out_specs=pl.BlockSpec((1, VOCAB), lambda: (0, 0)),
        out_shape=jax.ShapeDtypeStruct((1, VOCAB), jnp.float32),
        interpret=interpret,
    )(partials, Wt, b2d)


def kernel(inputs, table, W, b):
    idx = inputs.astype(jnp.int32)
    partials = _sc_gather_pool(idx, table.T)
    return _tc_head(partials, W.T, b.reshape(1, VOCAB))
